# baseline (speedup 1.0000x reference)
import functools

import jax
import jax.numpy as jnp
from jax import lax
from jax.experimental import pallas as pl
from jax.experimental.pallas import tpu as pltpu
from jax.experimental.pallas import tpu_sc as plsc

_INFO = plsc.get_sparse_core_info()
_NC, _NS = _INFO.num_cores, _INFO.num_subcores
_NW = _NC * _NS


@functools.lru_cache(maxsize=None)
def _make_gather(V, B, S, D):
    SP = ((S + 7) // 8) * 8
    nb = B // _NW
    G = 16
    n_grp = nb // G
    mesh = plsc.VectorSubcoreMesh(core_axis_name="c", subcore_axis_name="s")

    @functools.partial(
        pl.kernel,
        mesh=mesh,
        compiler_params=pltpu.CompilerParams(use_tc_tiling_on_sc=False),
        out_type=jax.ShapeDtypeStruct((B, S, D), jnp.float32),
        scratch_types=[
            pltpu.VMEM((nb, SP), jnp.int32),
            pltpu.VMEM((nb, SP, D), jnp.float32),
            pltpu.SemaphoreType.DMA,
        ],
    )
    def gather_kernel(idx_hbm, table_hbm, out_hbm, idx_v, slots, sem):
        wid = lax.axis_index("s") * _NC + lax.axis_index("c")
        pltpu.sync_copy(idx_hbm.at[pl.ds(wid * nb, nb)], idx_v)
        for g in range(n_grp):
            gathers = []
            for b in range(G):
                i = g * G + b
                gathers.append(
                    pltpu.async_copy(
                        table_hbm.at[idx_v.at[i]],
                        slots.at[i],
                        sem,
                    )
                )
            for c in gathers:
                c.wait()
        pltpu.sync_copy(
            slots.at[pl.ds(0, nb), pl.ds(0, S)],
            out_hbm.at[pl.ds(wid * nb, nb)],
        )

    return gather_kernel


def kernel(input_ids, table):
    bsz, seq = input_ids.shape
    vocab, dim = table.shape
    seq_pad = ((seq + 7) // 8) * 8
    idx_pad = jnp.pad(input_ids.astype(jnp.int32), ((0, 0), (0, seq_pad - seq)))
    h = _make_gather(vocab, bsz, seq, dim)(idx_pad, table)
    logits = jnp.zeros((bsz, seq, vocab), dtype=h.dtype)
    return (h, logits)


# speedup vs baseline: 1.8581x; 1.8581x over previous
import functools

import jax
import jax.numpy as jnp
from jax import lax
from jax.experimental import pallas as pl
from jax.experimental.pallas import tpu as pltpu
from jax.experimental.pallas import tpu_sc as plsc

_INFO = plsc.get_sparse_core_info()
_NC, _NS = _INFO.num_cores, _INFO.num_subcores
_NW = _NC * _NS


@functools.lru_cache(maxsize=None)
def _make_gather(V, D, B):
    assert B % _NW == 0
    b_per_w = B // _NW
    ch = 80
    assert b_per_w % ch == 0
    n_ch = b_per_w // ch
    mesh = plsc.VectorSubcoreMesh(core_axis_name="c", subcore_axis_name="s")

    @functools.partial(
        pl.kernel,
        mesh=mesh,
        compiler_params=pltpu.CompilerParams(use_tc_tiling_on_sc=False),
        out_type=jax.ShapeDtypeStruct((B, D), jnp.float32),
        cost_estimate=pl.CostEstimate(
            flops=0, bytes_accessed=3 * B * D * 4, transcendentals=0
        ),
        scratch_types=[
            pltpu.VMEM((b_per_w,), jnp.int32),
            pltpu.VMEM((b_per_w, D), jnp.float32),
            pltpu.SemaphoreType.DMA,
        ],
    )
    def gather_kernel(idx_hbm, table_hbm, out_hbm, idx_v, rows_v, sem):
        wid = lax.axis_index("s") * _NC + lax.axis_index("c")
        pltpu.sync_copy(idx_hbm.at[pl.ds(wid * b_per_w, b_per_w)], idx_v)
        copies = []
        for j in range(n_ch):
            copies.append(
                pltpu.async_copy(
                    table_hbm.at[idx_v.at[pl.ds(j * ch, ch)]],
                    rows_v.at[pl.ds(j * ch, ch)],
                    sem,
                )
            )
        for c in copies:
            c.wait()
        pltpu.sync_copy(rows_v, out_hbm.at[pl.ds(wid * b_per_w, b_per_w)])

    return gather_kernel


def kernel(input_ids, table):
    bsz, seq = input_ids.shape
    vocab, dim = table.shape
    flat = input_ids.reshape(-1).astype(jnp.int32)
    total = bsz * seq
    gathered = _make_gather(vocab, dim, total)(flat, table)
    h = gathered.reshape(bsz, seq, dim)
    logits = jnp.zeros((bsz, seq, vocab), dtype=h.dtype)
    return (h, logits)
